# baseline (device time: 34957 ns/iter reference)
import os

import jax
import jax.numpy as jnp
from jax import lax
from jax.experimental import pallas as pl
from jax.experimental.pallas import tpu as pltpu

N_DEV = 32
K_CHUNKS = 8

_KVAR = os.environ.get("KVAR", "full")
_EMPTY = _KVAR == "empty"
_DO_COMM = "nocomm" not in _KVAR and _KVAR not in ("nothing", "empty")
_DO_GEMM = "nogemm" not in _KVAR and _KVAR not in ("nothing", "empty")
_DO_DOT = "nodot" not in _KVAR
_DO_GELU = "nogelu" not in _KVAR
_DO_XCONV = "noxconv" not in _KVAR


def kernel(x, w_mat):
    m_total, k_shard = x.shape
    k_total, n_out = w_mat.shape
    m_blk = m_total // N_DEV
    kc = k_total // K_CHUNKS

    def body(x_ref, w_ref, out_ref,
             gathered_ref, wbuf_ref,
             send_sems, recv_sems, copy_sems):
        my = lax.axis_index("i")

        with jax.named_scope("barrier_and_prefetch"):
            if _DO_COMM:
                barrier_sem = pltpu.get_barrier_semaphore()
                for s in range(1, N_DEV):
                    peer = lax.rem(my + s, N_DEV)
                    pl.semaphore_signal(
                        barrier_sem, inc=1,
                        device_id=(peer,), device_id_type=pl.DeviceIdType.MESH,
                    )

            w_copies = [None] * K_CHUNKS
            if _DO_GEMM:
                w_copies[0] = pltpu.make_async_copy(
                    w_ref.at[pl.ds(0, kc), :], wbuf_ref.at[0], copy_sems.at[0],
                )
                w_copies[0].start()

            if _DO_COMM:
                pl.semaphore_wait(barrier_sem, N_DEV - 1)

        with jax.named_scope("a2a_send"):
            gathered_ref[:, pl.ds(my * k_shard, k_shard)] = (
                x_ref[pl.ds(my * m_blk, m_blk), :]
            )

            sends = []
            for s in range(1, N_DEV) if _DO_COMM else ():
                tgt = lax.rem(my + s, N_DEV)
                rdma = pltpu.make_async_remote_copy(
                    src_ref=x_ref.at[pl.ds(tgt * m_blk, m_blk), :],
                    dst_ref=gathered_ref.at[:, pl.ds(my * k_shard, k_shard)],
                    send_sem=send_sems.at[s],
                    recv_sem=recv_sems.at[s],
                    device_id=(tgt,),
                    device_id_type=pl.DeviceIdType.MESH,
                )
                rdma.start()
                sends.append(rdma)

        with jax.named_scope("a2a_wait_recv"):
            for s in range(1, N_DEV) if _DO_COMM else ():
                src = lax.rem(my + (N_DEV - s), N_DEV)
                recv = pltpu.make_async_remote_copy(
                    src_ref=x_ref.at[pl.ds(0, m_blk), :],
                    dst_ref=gathered_ref.at[:, pl.ds(src * k_shard, k_shard)],
                    send_sem=send_sems.at[s],
                    recv_sem=recv_sems.at[s],
                    device_id=(src,),
                    device_id_type=pl.DeviceIdType.MESH,
                )
                recv.wait_recv()

        acc = jnp.zeros((m_blk, n_out), dtype=jnp.float32)
        for c in range(K_CHUNKS) if _DO_GEMM else ():
            with jax.named_scope(f"gemm#chunk={c}"):
                if c + 1 < K_CHUNKS:
                    w_copies[c + 1] = pltpu.make_async_copy(
                        w_ref.at[pl.ds((c + 1) * kc, kc), :],
                        wbuf_ref.at[(c + 1) % 2],
                        copy_sems.at[(c + 1) % 2],
                    )
                    w_copies[c + 1].start()
                w_copies[c].wait()
                if _DO_DOT:
                    wbf = wbuf_ref[c % 2, :, :].astype(jnp.bfloat16)
                    acc = acc + jnp.dot(
                        gathered_ref[:, pl.ds(c * kc, kc)], wbf,
                        preferred_element_type=jnp.float32,
                    )

        with jax.named_scope("epilogue"):
            if _EMPTY or not _DO_GELU:
                out_ref[:, :] = acc
            else:
                g = 0.7978845608028654
                out_ref[:, :] = 0.5 * acc * (
                    1.0 + jnp.tanh(g * (acc + 0.044715 * acc * acc * acc))
                )

            for rdma in sends:
                rdma.wait_send()

    x = x.astype(jnp.bfloat16)

    return pl.pallas_call(
        body,
        out_shape=jax.ShapeDtypeStruct((m_blk, n_out), jnp.float32),
        in_specs=[
            pl.BlockSpec(memory_space=pl.ANY if _EMPTY else pltpu.VMEM),
            pl.BlockSpec(memory_space=pl.ANY),
        ],
        out_specs=pl.BlockSpec(memory_space=pltpu.VMEM),
        scratch_shapes=[
            pltpu.VMEM((m_blk, k_total), jnp.bfloat16),
            pltpu.VMEM((2, kc, n_out), jnp.float32),
            pltpu.SemaphoreType.DMA((N_DEV,)),
            pltpu.SemaphoreType.DMA((N_DEV,)),
            pltpu.SemaphoreType.DMA((2,)),
        ],
        compiler_params=pltpu.CompilerParams(
            collective_id=0 if _DO_COMM else None,
        ),
    )(x, w_mat)


# device time: 29655 ns/iter; 1.1788x vs baseline; 1.1788x over previous
import jax
import jax.numpy as jnp
from jax import lax
from jax.experimental import pallas as pl
from jax.experimental.pallas import tpu as pltpu

N_DEV = 32
GROUP = 4
N_GROUPS = N_DEV // GROUP


def kernel(x, w_mat):
    m_total, k_shard = x.shape
    k_total, n_out = w_mat.shape
    m_blk = m_total // N_DEV
    kc = GROUP * k_shard

    def body(x_ref, w_ref, out_ref,
             slots_ref, gathered_ref, wbuf_ref,
             send_sems, recv_sems, copy_sems, loc_sems):
        my = lax.axis_index("i")

        barrier_sem = pltpu.get_barrier_semaphore()
        for s in range(1, N_DEV):
            peer = lax.rem(my + s, N_DEV)
            pl.semaphore_signal(
                barrier_sem, inc=1,
                device_id=(peer,), device_id_type=pl.DeviceIdType.MESH,
            )
        pl.semaphore_wait(barrier_sem, N_DEV - 1)

        gathered_ref[:, pl.ds(0, k_shard)] = x_ref[pl.ds(my * m_blk, m_blk), :]

        sends = []
        for s in range(1, N_DEV):
            tgt = lax.rem(my + s, N_DEV)
            rdma = pltpu.make_async_remote_copy(
                src_ref=x_ref.at[pl.ds(tgt * m_blk, m_blk), :],
                dst_ref=slots_ref.at[s],
                send_sem=send_sems.at[s],
                recv_sem=recv_sems.at[s],
                device_id=(tgt,),
                device_id_type=pl.DeviceIdType.MESH,
            )
            rdma.start()
            sends.append(rdma)

        def start_w_group(j):
            copies = []
            for i in range(GROUP):
                t = j * GROUP + i
                src = lax.rem(my + (N_DEV - t), N_DEV)
                cp = pltpu.make_async_copy(
                    w_ref.at[pl.ds(src * k_shard, k_shard), :],
                    wbuf_ref.at[j % 2, pl.ds(i * k_shard, k_shard), :],
                    copy_sems.at[j % 2],
                )
                cp.start()
                copies.append(cp)
            return copies

        w_copies = start_w_group(0)

        acc = jnp.zeros((m_blk, n_out), dtype=jnp.float32)
        for j in range(N_GROUPS):
            nxt = start_w_group(j + 1) if j + 1 < N_GROUPS else []

            loc_copies = []
            for i in range(GROUP):
                t = j * GROUP + i
                if t == 0:
                    continue
                recv = pltpu.make_async_remote_copy(
                    src_ref=x_ref.at[pl.ds(0, m_blk), :],
                    dst_ref=slots_ref.at[t],
                    send_sem=send_sems.at[t],
                    recv_sem=recv_sems.at[t],
                    device_id=(0,),
                    device_id_type=pl.DeviceIdType.MESH,
                )
                recv.wait_recv()
                lc = pltpu.make_async_copy(
                    slots_ref.at[t],
                    gathered_ref.at[:, pl.ds(t * k_shard, k_shard)],
                    loc_sems.at[i],
                )
                lc.start()
                loc_copies.append(lc)

            for cp in w_copies:
                cp.wait()
            for lc in loc_copies:
                lc.wait()
            w_copies = nxt

            wbf = wbuf_ref[j % 2, :, :].astype(jnp.bfloat16)
            acc = acc + jnp.dot(
                gathered_ref[:, pl.ds(j * kc, kc)], wbf,
                preferred_element_type=jnp.float32,
            )

        g = 0.7978845608028654
        out_ref[:, :] = 0.5 * acc * (
            1.0 + jnp.tanh(g * (acc + 0.044715 * acc * acc * acc))
        )

        for rdma in sends:
            rdma.wait_send()

    x = x.astype(jnp.bfloat16)

    return pl.pallas_call(
        body,
        out_shape=jax.ShapeDtypeStruct((m_blk, n_out), jnp.float32),
        in_specs=[
            pl.BlockSpec(memory_space=pltpu.VMEM),
            pl.BlockSpec(memory_space=pl.ANY),
        ],
        out_specs=pl.BlockSpec(memory_space=pltpu.VMEM),
        scratch_shapes=[
            pltpu.VMEM((N_DEV, m_blk, k_shard), jnp.bfloat16),
            pltpu.VMEM((m_blk, k_total), jnp.bfloat16),
            pltpu.VMEM((2, kc, n_out), jnp.float32),
            pltpu.SemaphoreType.DMA((N_DEV,)),
            pltpu.SemaphoreType.DMA((N_DEV,)),
            pltpu.SemaphoreType.DMA((2,)),
            pltpu.SemaphoreType.DMA((GROUP,)),
        ],
        compiler_params=pltpu.CompilerParams(collective_id=0),
    )(x, w_mat)
